# Initial kernel scaffold; baseline (speedup 1.0000x reference)
import jax
import jax.numpy as jnp
from jax.experimental import pallas as pl


def kernel(input, embed, pos_weight):
    raise NotImplementedError("write your pallas kernel here")


# speedup vs baseline: 1.8951x; 1.8951x over previous
import jax
import jax.numpy as jnp
from jax.experimental import pallas as pl

_DIM = 16
_N_EMBED = 1024
_ROWS = 1024
_TOTAL_ROWS = 8 * 512 * 16
_POS_ROWS = 512 * 16
_COMMITMENT = 1.0


def _vq_block(x_ref, pos_ref, embed_ref, q_ref, ind_ref, loss_ref):
    i = pl.program_id(0)
    x = x_ref[...]
    flat = x + pos_ref[...]
    emb = embed_ref[...]

    fl2 = jnp.sum(flat * flat, axis=1, keepdims=True)
    e2 = jnp.sum(emb * emb, axis=0, keepdims=True)
    mm = jnp.dot(flat, emb, preferred_element_type=jnp.float32)
    neg = -((fl2 - 2.0 * mm) + e2)

    m = jnp.max(neg, axis=1, keepdims=True)
    ids = jax.lax.broadcasted_iota(jnp.int32, (_ROWS, _N_EMBED), 1)
    idx = jnp.min(jnp.where(neg == m, ids, _N_EMBED), axis=1)

    onehot = (ids == idx[:, None]).astype(jnp.float32)
    quant = jax.lax.dot_general(
        onehot, emb, (((1,), (1,)), ((), ())),
        preferred_element_type=jnp.float32)

    q_ref[...] = quant
    ind_ref[...] = idx.reshape(1, 1, _ROWS)

    part = jnp.sum((quant - x) ** 2)

    @pl.when(i == 0)
    def _init():
        loss_ref[...] = jnp.zeros_like(loss_ref)

    loss_ref[...] += part


def kernel(input, embed, pos_weight):
    b, c, h, w = input.shape
    x2d = input.reshape(_TOTAL_ROWS, _DIM)
    pos2d = pos_weight.reshape(_POS_ROWS, _DIM)
    grid = _TOTAL_ROWS // _ROWS
    pos_blocks = _POS_ROWS // _ROWS

    q2d, ind, loss_sum = pl.pallas_call(
        _vq_block,
        grid=(grid,),
        in_specs=[
            pl.BlockSpec((_ROWS, _DIM), lambda i: (i, 0)),
            pl.BlockSpec((_ROWS, _DIM), lambda i: (i % pos_blocks, 0)),
            pl.BlockSpec((_DIM, _N_EMBED), lambda i: (0, 0)),
        ],
        out_specs=[
            pl.BlockSpec((_ROWS, _DIM), lambda i: (i, 0)),
            pl.BlockSpec((1, 1, _ROWS), lambda i: (i, 0, 0)),
            pl.BlockSpec((1, 1), lambda i: (0, 0)),
        ],
        out_shape=[
            jax.ShapeDtypeStruct((_TOTAL_ROWS, _DIM), jnp.float32),
            jax.ShapeDtypeStruct((grid, 1, _ROWS), jnp.int32),
            jax.ShapeDtypeStruct((1, 1), jnp.float32),
        ],
    )(x2d, pos2d, embed)

    quantize = q2d.reshape(b, c, h, w)
    embed_ind = ind.reshape(b, c, _DIM)
    loss = loss_sum[0, 0] / jnp.float32(_TOTAL_ROWS * _DIM) * _COMMITMENT
    return quantize, embed_ind, loss
